# baseline (speedup 1.0000x reference)
import functools

import jax
import jax.numpy as jnp
from jax.experimental import pallas as pl
from jax.experimental.pallas import tpu as pltpu


def _fused_kernel_t(x_ref, w1t_ref, b1_ref, w2_ref, b2_ref, o_ref,
                    stash_ref, accs_ref, accm_ref, scale_ref, *,
                    inv_s, k_in, st_in, st_out):
    k = pl.program_id(0)

    @pl.when(k < k_in)
    def _():
        x = x_ref[...].astype(jnp.float32)
        ps = jnp.sum(x, axis=1)
        pm = jnp.max(x, axis=1)

        @pl.when(k == 0)
        def _():
            accs_ref[...] = ps
            accm_ref[...] = pm

        @pl.when(k != 0)
        def _():
            accs_ref[...] = accs_ref[...] + ps
            accm_ref[...] = jnp.maximum(accm_ref[...], pm)

        stash_ref[:, pl.ds(k * st_in, st_in), :] = x.astype(jnp.bfloat16)

        @pl.when(k == k_in - 1)
        def _():
            avg = accs_ref[...] * inv_s
            mx = accm_ref[...]
            w1t = w1t_ref[...]
            w2 = w2_ref[...]
            b1 = b1_ref[...].reshape(-1, 1)
            b2 = b2_ref[...].reshape(-1, 1)

            def mlp(p):
                h = jax.lax.dot_general(
                    w1t, p, (((1,), (0,)), ((), ())),
                    preferred_element_type=jnp.float32)
                h = jnp.maximum(h + b1, 0.0)
                o = jax.lax.dot_general(
                    w2, h, (((0,), (0,)), ((), ())),
                    preferred_element_type=jnp.float32)
                return o + b2

            scale_ref[...] = jax.nn.sigmoid(mlp(avg) + mlp(mx))

    @pl.when(k >= k_in)
    def _():
        j = k - k_in
        xb = stash_ref[:, pl.ds(j * st_out, st_out), :].astype(jnp.float32)
        o_ref[...] = (xb * scale_ref[...][:, None, :]).astype(o_ref.dtype)


def _channel_gate_fused(x3, w1, b1, w2, b2, S, st_in, st_out):
    B, C, _ = x3.shape
    xT = jnp.transpose(x3, (1, 2, 0))
    w1t = jnp.transpose(w1)

    k_in = S // st_in
    k_out = S // st_out

    outT = pl.pallas_call(
        functools.partial(_fused_kernel_t, inv_s=1.0 / S,
                          k_in=k_in, st_in=st_in, st_out=st_out),
        out_shape=jax.ShapeDtypeStruct((C, S, B), x3.dtype),
        grid=(k_in + k_out,),
        in_specs=[
            pl.BlockSpec((C, st_in, B),
                         lambda k: (0, jnp.where(k < k_in, k, k_in - 1), 0)),
            pl.BlockSpec(w1t.shape, lambda k: (0, 0)),
            pl.BlockSpec(b1.shape, lambda k: (0, 0)),
            pl.BlockSpec(w2.shape, lambda k: (0, 0)),
            pl.BlockSpec(b2.shape, lambda k: (0, 0)),
        ],
        out_specs=pl.BlockSpec(
            (C, st_out, B),
            lambda k: (0, jnp.where(k < k_in, 0, k - k_in), 0)),
        scratch_shapes=[
            pltpu.VMEM((C, S, B), jnp.bfloat16),
            pltpu.VMEM((C, B), jnp.float32),
            pltpu.VMEM((C, B), jnp.float32),
            pltpu.VMEM((C, B), jnp.float32),
        ],
        compiler_params=pltpu.CompilerParams(
            dimension_semantics=("arbitrary",)
        ),
    )(xT, w1t, b1, w2, b2)

    return jnp.transpose(outT, (2, 0, 1))


def _pool_kernel_t(x_ref, sum_ref, max_ref):
    k = pl.program_id(1)
    x = x_ref[...].astype(jnp.float32)
    ps = jnp.sum(x, axis=1)
    pm = jnp.max(x, axis=1)

    @pl.when(k == 0)
    def _():
        sum_ref[0] = ps
        max_ref[0] = pm

    @pl.when(k != 0)
    def _():
        sum_ref[0] = sum_ref[0] + ps
        max_ref[0] = jnp.maximum(max_ref[0], pm)


def _apply_kernel_t(x_ref, psum_ref, pmax_ref, w1t_ref, b1_ref, w2_ref,
                    b2_ref, o_ref, *, inv_s):
    s = jnp.sum(psum_ref[...], axis=0)
    m = jnp.max(pmax_ref[...], axis=0)
    avg = s * inv_s

    w1t = w1t_ref[...]
    w2 = w2_ref[...]
    b1 = b1_ref[...].reshape(-1, 1)
    b2 = b2_ref[...].reshape(-1, 1)

    def mlp(p):
        h = jax.lax.dot_general(
            w1t, p, (((1,), (0,)), ((), ())),
            preferred_element_type=jnp.float32)
        h = jnp.maximum(h + b1, 0.0)
        o = jax.lax.dot_general(
            w2, h, (((0,), (0,)), ((), ())),
            preferred_element_type=jnp.float32)
        return o + b2

    scale = jax.nn.sigmoid(mlp(avg) + mlp(m))
    o_ref[...] = (x_ref[...] * scale[:, None, :].astype(o_ref.dtype))


def _channel_gate_native(x3, w1, b1, w2, b2, S):
    B, C, _ = x3.shape
    xT = jnp.transpose(x3, (1, 2, 0))

    ST = next(t for t in (512, 256, 128, 64, 32, 16, 8) if S % t == 0)
    N = S // ST
    P = 2 if N % 2 == 0 else 1
    K = N // P

    STp = next(t for t in (1024, 512, 256, 128, 64, 32, 16, 8)
               if S % t == 0)
    Np = S // STp
    Pp = 2 if Np % 2 == 0 else 1
    Kp = Np // Pp

    psum, pmax = pl.pallas_call(
        _pool_kernel_t,
        out_shape=(
            jax.ShapeDtypeStruct((Pp, C, B), jnp.float32),
            jax.ShapeDtypeStruct((Pp, C, B), jnp.float32),
        ),
        grid=(Pp, Kp),
        in_specs=[pl.BlockSpec((C, STp, B), lambda p, k: (0, p * Kp + k, 0))],
        out_specs=(
            pl.BlockSpec((1, C, B), lambda p, k: (p, 0, 0)),
            pl.BlockSpec((1, C, B), lambda p, k: (p, 0, 0)),
        ),
        compiler_params=pltpu.CompilerParams(
            dimension_semantics=("parallel", "arbitrary")
        ),
    )(xT)

    w1t = jnp.transpose(w1)

    outT = pl.pallas_call(
        functools.partial(_apply_kernel_t, inv_s=1.0 / S),
        out_shape=jax.ShapeDtypeStruct((C, S, B), x3.dtype),
        grid=(P, K),
        in_specs=[
            pl.BlockSpec((C, ST, B), lambda p, k: (0, p * K + k, 0)),
            pl.BlockSpec((Pp, C, B), lambda p, k: (0, 0, 0)),
            pl.BlockSpec((Pp, C, B), lambda p, k: (0, 0, 0)),
            pl.BlockSpec(w1t.shape, lambda p, k: (0, 0)),
            pl.BlockSpec(b1.shape, lambda p, k: (0, 0)),
            pl.BlockSpec(w2.shape, lambda p, k: (0, 0)),
            pl.BlockSpec(b2.shape, lambda p, k: (0, 0)),
        ],
        out_specs=pl.BlockSpec((C, ST, B), lambda p, k: (0, p * K + k, 0)),
        compiler_params=pltpu.CompilerParams(
            dimension_semantics=("parallel", "parallel")
        ),
    )(xT, psum, pmax, w1t, b1, w2, b2)

    return jnp.transpose(outT, (2, 0, 1))


def _gate_kernel(x_ref, w1_ref, b1_ref, w2_ref, b2_ref, o_ref, *,
                 s_true, needs_mask):
    x = x_ref[...].astype(jnp.float32)

    if needs_mask:
        lane = jax.lax.broadcasted_iota(jnp.int32, x.shape, 2)
        x_for_max = jnp.where(lane < s_true, x, -jnp.inf)
    else:
        x_for_max = x

    avg = jnp.sum(x, axis=-1) * (1.0 / s_true)
    mx = jnp.max(x_for_max, axis=-1)

    def mlp(p):
        h = jnp.maximum(
            jnp.dot(p, w1_ref[...], preferred_element_type=jnp.float32)
            + b1_ref[...], 0.0)
        return jnp.dot(h, w2_ref[...],
                       preferred_element_type=jnp.float32) + b2_ref[...]

    scale = jax.nn.sigmoid(mlp(avg) + mlp(mx))
    o_ref[...] = (x * scale[:, :, None]).astype(o_ref.dtype)


def _channel_gate_padded(x3, w1, b1, w2, b2, S):
    B, C, _ = x3.shape
    LANE = 128
    s_pad = -(-S // LANE) * LANE
    if s_pad != S:
        x3 = jnp.pad(x3, ((0, 0), (0, 0), (0, s_pad - S)))

    BT = 8
    while B % BT != 0:
        BT //= 2

    out3 = pl.pallas_call(
        functools.partial(_gate_kernel, s_true=S, needs_mask=(s_pad != S)),
        out_shape=jax.ShapeDtypeStruct((B, C, s_pad), x3.dtype),
        grid=(B // BT,),
        in_specs=[
            pl.BlockSpec((BT, C, s_pad), lambda i: (i, 0, 0)),
            pl.BlockSpec(w1.shape, lambda i: (0, 0)),
            pl.BlockSpec(b1.shape, lambda i: (0, 0)),
            pl.BlockSpec(w2.shape, lambda i: (0, 0)),
            pl.BlockSpec(b2.shape, lambda i: (0, 0)),
        ],
        out_specs=pl.BlockSpec((BT, C, s_pad), lambda i: (i, 0, 0)),
        compiler_params=pltpu.CompilerParams(
            dimension_semantics=("parallel",)
        ),
    )(x3, w1, b1, w2, b2)

    return out3[:, :, :S]


def kernel(x, w1, b1, w2, b2):
    B, C, D, H, W = x.shape
    S = D * H * W
    x3 = x.reshape(B, C, S)
    if S % 8 == 0:
        lanes = -(-B // 128) * 128
        st_in = next(t for t in (256, 128, 64, 32, 16, 8) if S % t == 0)
        st_out = next(t for t in (256, 128, 64, 32, 16, 8) if S % t == 0)
        vmem_bytes = (C * S * lanes * 2
                      + 2 * C * (st_in + st_out) * lanes * 4)
        if vmem_bytes <= 52 * 1024 * 1024:
            out3 = _channel_gate_fused(x3, w1, b1, w2, b2, S, st_in, st_out)
        else:
            out3 = _channel_gate_native(x3, w1, b1, w2, b2, S)
    else:
        out3 = _channel_gate_padded(x3, w1, b1, w2, b2, S)
    return out3.reshape(B, C, D, H, W)


# speedup vs baseline: 1.0073x; 1.0073x over previous
import functools

import jax
import jax.numpy as jnp
from jax.experimental import pallas as pl
from jax.experimental.pallas import tpu as pltpu


def _fused_kernel_t(x_ref, w1t_ref, b1_ref, w2_ref, b2_ref, o_ref,
                    stash_ref, accs_ref, accm_ref, scale_ref, *,
                    inv_s, k_in, st_in, st_out):
    k = pl.program_id(0)

    @pl.when(k < k_in)
    def _():
        x = x_ref[...].astype(jnp.float32)
        ps = jnp.sum(x, axis=1)
        pm = jnp.max(x, axis=1)

        @pl.when(k == 0)
        def _():
            accs_ref[...] = ps
            accm_ref[...] = pm

        @pl.when(k != 0)
        def _():
            accs_ref[...] = accs_ref[...] + ps
            accm_ref[...] = jnp.maximum(accm_ref[...], pm)

        @pl.when(k < k_in - 1)
        def _():
            stash_ref[:, pl.ds(k * st_in, st_in), :] = x.astype(jnp.bfloat16)

        @pl.when(k == k_in - 1)
        def _():
            avg = accs_ref[...] * inv_s
            mx = accm_ref[...]
            w1t = w1t_ref[...]
            w2 = w2_ref[...]
            b1 = b1_ref[...].reshape(-1, 1)
            b2 = b2_ref[...].reshape(-1, 1)

            def mlp(p):
                h = jax.lax.dot_general(
                    w1t, p, (((1,), (0,)), ((), ())),
                    preferred_element_type=jnp.float32)
                h = jnp.maximum(h + b1, 0.0)
                o = jax.lax.dot_general(
                    w2, h, (((0,), (0,)), ((), ())),
                    preferred_element_type=jnp.float32)
                return o + b2

            scale_ref[...] = jax.nn.sigmoid(mlp(avg) + mlp(mx))

    j_boundary = (k_in - 1) * st_in // st_out

    @pl.when(k >= k_in)
    def _():
        j = k - k_in
        scale = scale_ref[...][:, None, :]

        @pl.when(j < j_boundary)
        def _():
            xb = stash_ref[:, pl.ds(j * st_out, st_out), :].astype(
                jnp.float32)
            o_ref[...] = (xb * scale).astype(o_ref.dtype)

        @pl.when(j >= j_boundary)
        def _():
            xb = x_ref[:, pl.ds((j - j_boundary) * st_out, st_out), :]
            o_ref[...] = (xb.astype(jnp.float32) * scale).astype(o_ref.dtype)


def _channel_gate_fused(x3, w1, b1, w2, b2, S, st_in, st_out):
    B, C, _ = x3.shape
    xT = jnp.transpose(x3, (1, 2, 0))
    w1t = jnp.transpose(w1)

    k_in = S // st_in
    k_out = S // st_out

    outT = pl.pallas_call(
        functools.partial(_fused_kernel_t, inv_s=1.0 / S,
                          k_in=k_in, st_in=st_in, st_out=st_out),
        out_shape=jax.ShapeDtypeStruct((C, S, B), x3.dtype),
        grid=(k_in + k_out,),
        in_specs=[
            pl.BlockSpec((C, st_in, B),
                         lambda k: (0, jnp.where(k < k_in, k, k_in - 1), 0)),
            pl.BlockSpec(w1t.shape, lambda k: (0, 0)),
            pl.BlockSpec(b1.shape, lambda k: (0, 0)),
            pl.BlockSpec(w2.shape, lambda k: (0, 0)),
            pl.BlockSpec(b2.shape, lambda k: (0, 0)),
        ],
        out_specs=pl.BlockSpec(
            (C, st_out, B),
            lambda k: (0, jnp.where(k < k_in, 0, k - k_in), 0)),
        scratch_shapes=[
            pltpu.VMEM((C, max(S - st_in, st_in), B), jnp.bfloat16),
            pltpu.VMEM((C, B), jnp.float32),
            pltpu.VMEM((C, B), jnp.float32),
            pltpu.VMEM((C, B), jnp.float32),
        ],
        compiler_params=pltpu.CompilerParams(
            dimension_semantics=("arbitrary",)
        ),
    )(xT, w1t, b1, w2, b2)

    return jnp.transpose(outT, (2, 0, 1))


def _pool_kernel_t(x_ref, sum_ref, max_ref):
    k = pl.program_id(1)
    x = x_ref[...].astype(jnp.float32)
    ps = jnp.sum(x, axis=1)
    pm = jnp.max(x, axis=1)

    @pl.when(k == 0)
    def _():
        sum_ref[0] = ps
        max_ref[0] = pm

    @pl.when(k != 0)
    def _():
        sum_ref[0] = sum_ref[0] + ps
        max_ref[0] = jnp.maximum(max_ref[0], pm)


def _apply_kernel_t(x_ref, psum_ref, pmax_ref, w1t_ref, b1_ref, w2_ref,
                    b2_ref, o_ref, *, inv_s):
    s = jnp.sum(psum_ref[...], axis=0)
    m = jnp.max(pmax_ref[...], axis=0)
    avg = s * inv_s

    w1t = w1t_ref[...]
    w2 = w2_ref[...]
    b1 = b1_ref[...].reshape(-1, 1)
    b2 = b2_ref[...].reshape(-1, 1)

    def mlp(p):
        h = jax.lax.dot_general(
            w1t, p, (((1,), (0,)), ((), ())),
            preferred_element_type=jnp.float32)
        h = jnp.maximum(h + b1, 0.0)
        o = jax.lax.dot_general(
            w2, h, (((0,), (0,)), ((), ())),
            preferred_element_type=jnp.float32)
        return o + b2

    scale = jax.nn.sigmoid(mlp(avg) + mlp(m))
    o_ref[...] = (x_ref[...] * scale[:, None, :].astype(o_ref.dtype))


def _channel_gate_native(x3, w1, b1, w2, b2, S):
    B, C, _ = x3.shape
    xT = jnp.transpose(x3, (1, 2, 0))

    ST = next(t for t in (512, 256, 128, 64, 32, 16, 8) if S % t == 0)
    N = S // ST
    P = 2 if N % 2 == 0 else 1
    K = N // P

    STp = next(t for t in (1024, 512, 256, 128, 64, 32, 16, 8)
               if S % t == 0)
    Np = S // STp
    Pp = 2 if Np % 2 == 0 else 1
    Kp = Np // Pp

    psum, pmax = pl.pallas_call(
        _pool_kernel_t,
        out_shape=(
            jax.ShapeDtypeStruct((Pp, C, B), jnp.float32),
            jax.ShapeDtypeStruct((Pp, C, B), jnp.float32),
        ),
        grid=(Pp, Kp),
        in_specs=[pl.BlockSpec((C, STp, B), lambda p, k: (0, p * Kp + k, 0))],
        out_specs=(
            pl.BlockSpec((1, C, B), lambda p, k: (p, 0, 0)),
            pl.BlockSpec((1, C, B), lambda p, k: (p, 0, 0)),
        ),
        compiler_params=pltpu.CompilerParams(
            dimension_semantics=("parallel", "arbitrary")
        ),
    )(xT)

    w1t = jnp.transpose(w1)

    outT = pl.pallas_call(
        functools.partial(_apply_kernel_t, inv_s=1.0 / S),
        out_shape=jax.ShapeDtypeStruct((C, S, B), x3.dtype),
        grid=(P, K),
        in_specs=[
            pl.BlockSpec((C, ST, B), lambda p, k: (0, p * K + k, 0)),
            pl.BlockSpec((Pp, C, B), lambda p, k: (0, 0, 0)),
            pl.BlockSpec((Pp, C, B), lambda p, k: (0, 0, 0)),
            pl.BlockSpec(w1t.shape, lambda p, k: (0, 0)),
            pl.BlockSpec(b1.shape, lambda p, k: (0, 0)),
            pl.BlockSpec(w2.shape, lambda p, k: (0, 0)),
            pl.BlockSpec(b2.shape, lambda p, k: (0, 0)),
        ],
        out_specs=pl.BlockSpec((C, ST, B), lambda p, k: (0, p * K + k, 0)),
        compiler_params=pltpu.CompilerParams(
            dimension_semantics=("parallel", "parallel")
        ),
    )(xT, psum, pmax, w1t, b1, w2, b2)

    return jnp.transpose(outT, (2, 0, 1))


def _gate_kernel(x_ref, w1_ref, b1_ref, w2_ref, b2_ref, o_ref, *,
                 s_true, needs_mask):
    x = x_ref[...].astype(jnp.float32)

    if needs_mask:
        lane = jax.lax.broadcasted_iota(jnp.int32, x.shape, 2)
        x_for_max = jnp.where(lane < s_true, x, -jnp.inf)
    else:
        x_for_max = x

    avg = jnp.sum(x, axis=-1) * (1.0 / s_true)
    mx = jnp.max(x_for_max, axis=-1)

    def mlp(p):
        h = jnp.maximum(
            jnp.dot(p, w1_ref[...], preferred_element_type=jnp.float32)
            + b1_ref[...], 0.0)
        return jnp.dot(h, w2_ref[...],
                       preferred_element_type=jnp.float32) + b2_ref[...]

    scale = jax.nn.sigmoid(mlp(avg) + mlp(mx))
    o_ref[...] = (x * scale[:, :, None]).astype(o_ref.dtype)


def _channel_gate_padded(x3, w1, b1, w2, b2, S):
    B, C, _ = x3.shape
    LANE = 128
    s_pad = -(-S // LANE) * LANE
    if s_pad != S:
        x3 = jnp.pad(x3, ((0, 0), (0, 0), (0, s_pad - S)))

    BT = 8
    while B % BT != 0:
        BT //= 2

    out3 = pl.pallas_call(
        functools.partial(_gate_kernel, s_true=S, needs_mask=(s_pad != S)),
        out_shape=jax.ShapeDtypeStruct((B, C, s_pad), x3.dtype),
        grid=(B // BT,),
        in_specs=[
            pl.BlockSpec((BT, C, s_pad), lambda i: (i, 0, 0)),
            pl.BlockSpec(w1.shape, lambda i: (0, 0)),
            pl.BlockSpec(b1.shape, lambda i: (0, 0)),
            pl.BlockSpec(w2.shape, lambda i: (0, 0)),
            pl.BlockSpec(b2.shape, lambda i: (0, 0)),
        ],
        out_specs=pl.BlockSpec((BT, C, s_pad), lambda i: (i, 0, 0)),
        compiler_params=pltpu.CompilerParams(
            dimension_semantics=("parallel",)
        ),
    )(x3, w1, b1, w2, b2)

    return out3[:, :, :S]


def kernel(x, w1, b1, w2, b2):
    B, C, D, H, W = x.shape
    S = D * H * W
    x3 = x.reshape(B, C, S)
    if S % 8 == 0:
        lanes = -(-B // 128) * 128
        st_in = next(t for t in (512, 256, 128, 64, 32, 16, 8) if S % t == 0)
        st_out = next(t for t in (128, 64, 32, 16, 8) if st_in % t == 0)
        vmem_bytes = (C * max(S - st_in, st_in) * lanes * 2
                      + 2 * C * (st_in + st_out) * lanes * 4)
        if vmem_bytes <= 52 * 1024 * 1024:
            out3 = _channel_gate_fused(x3, w1, b1, w2, b2, S, st_in, st_out)
        else:
            out3 = _channel_gate_native(x3, w1, b1, w2, b2, S)
    else:
        out3 = _channel_gate_padded(x3, w1, b1, w2, b2, S)
    return out3.reshape(B, C, D, H, W)
